# baseline (speedup 1.0000x reference)
import functools

import jax
import jax.numpy as jnp
from jax.experimental import pallas as pl
from jax.experimental.pallas import tpu as pltpu


def _add_kernel(bt, x_ref, emb_ref, o_ref):
    t = pl.program_id(1)
    o_ref[...] = x_ref[...] + emb_ref[pl.ds(t * bt, bt), :]


def kernel(x, embed_table):
    B, T, D = x.shape
    bt = 1024
    grid = (B, T // bt)
    return pl.pallas_call(
        functools.partial(_add_kernel, bt),
        grid=grid,
        in_specs=[
            pl.BlockSpec((1, bt, D), lambda b, t: (b, t, 0)),
            pl.BlockSpec((T, D), lambda b, t: (0, 0)),
        ],
        out_specs=pl.BlockSpec((1, bt, D), lambda b, t: (b, t, 0)),
        out_shape=jax.ShapeDtypeStruct((B, T, D), x.dtype),
        compiler_params=pltpu.CompilerParams(
            dimension_semantics=("parallel", "parallel"),
            vmem_limit_bytes=128 * 1024 * 1024,
        ),
    )(x, embed_table)


# speedup vs baseline: 1.0340x; 1.0340x over previous
import functools

import jax
import jax.numpy as jnp
from jax.experimental import pallas as pl
from jax.experimental.pallas import tpu as pltpu


def _add_kernel(bt, x_ref, emb_ref, o_ref):
    t = pl.program_id(1)
    o_ref[...] = x_ref[...] + emb_ref[pl.ds(t * bt, bt), :]


def kernel(x, embed_table):
    B, T, D = x.shape
    bt = 2048
    grid = (B, T // bt)
    return pl.pallas_call(
        functools.partial(_add_kernel, bt),
        grid=grid,
        in_specs=[
            pl.BlockSpec((1, bt, D), lambda b, t: (b, t, 0)),
            pl.BlockSpec((T, D), lambda b, t: (0, 0)),
        ],
        out_specs=pl.BlockSpec((1, bt, D), lambda b, t: (b, t, 0)),
        out_shape=jax.ShapeDtypeStruct((B, T, D), x.dtype),
        compiler_params=pltpu.CompilerParams(
            dimension_semantics=("parallel", "parallel"),
            vmem_limit_bytes=128 * 1024 * 1024,
        ),
    )(x, embed_table)
